# baseline (speedup 1.0000x reference)
import functools
import math

import jax
import jax.numpy as jnp
from jax import lax
from jax.experimental import pallas as pl
from jax.experimental.pallas import tpu as pltpu
from jax.experimental.pallas import tpu_sc as plsc

_VOCAB = 100000
_MAXLEN = 2048
_HID = 128
_BATCH = 64

_INFO = plsc.get_sparse_core_info()
_NC = _INFO.num_cores
_NS = _INFO.num_subcores
_NW = _NC * _NS
_TPW = _MAXLEN // _NW
_LANES = _INFO.num_lanes
_SCALE = math.sqrt(float(_HID))


def _body(x_hbm, tok_hbm, pos_hbm, out_hbm, idx_v, rows_v, pos_v, sem):
    wid = lax.axis_index("s") * _NC + lax.axis_index("c")
    t0 = wid * _TPW

    pltpu.sync_copy(pos_hbm.at[pl.ds(t0, _TPW)], pos_v)

    def step(b, carry):
        pltpu.sync_copy(x_hbm.at[b, pl.ds(t0, _TPW)], idx_v)
        pltpu.async_copy(tok_hbm.at[idx_v], rows_v, sem).wait()

        def fma_row(r, c):
            for j in range(_HID // _LANES):
                sl = pl.ds(j * _LANES, _LANES)
                rows_v[r, sl] = rows_v[r, sl] * _SCALE + pos_v[r, sl]
            return c

        lax.fori_loop(0, _TPW, fma_row, 0)
        pltpu.sync_copy(rows_v, out_hbm.at[b, pl.ds(t0, _TPW)])
        return carry

    lax.fori_loop(0, _BATCH, step, 0)


@jax.jit
def kernel(x, token_table, pos_table):
    x = x.astype(jnp.int32)
    mesh = plsc.VectorSubcoreMesh(core_axis_name="c", subcore_axis_name="s")
    f = functools.partial(
        pl.kernel,
        mesh=mesh,
        out_type=jax.ShapeDtypeStruct((_BATCH, _MAXLEN, _HID), jnp.float32),
        scratch_types=[
            pltpu.VMEM((_TPW,), jnp.int32),
            pltpu.VMEM((_TPW, _HID), jnp.float32),
            pltpu.VMEM((_TPW, _HID), jnp.float32),
            pltpu.SemaphoreType.DMA,
        ],
    )(_body)
    return f(x, token_table, pos_table)


# speedup vs baseline: 2.5063x; 2.5063x over previous
import functools
import math

import jax
import jax.numpy as jnp
from jax import lax
from jax.experimental import pallas as pl
from jax.experimental.pallas import tpu as pltpu
from jax.experimental.pallas import tpu_sc as plsc

_VOCAB = 100000
_MAXLEN = 2048
_HID = 128
_BATCH = 64

_INFO = plsc.get_sparse_core_info()
_NC = _INFO.num_cores
_NS = _INFO.num_subcores
_NW = _NC * _NS
_TPW = _MAXLEN // _NW
_LANES = _INFO.num_lanes
_SCALE = math.sqrt(float(_HID))
_NBUF = 4


def _body(x_hbm, tok_hbm, pos_hbm, out_hbm, idx_v, gbuf, obuf, pos_v,
          gsems, ssems):
    wid = lax.axis_index("s") * _NC + lax.axis_index("c")
    t0 = wid * _TPW
    c0 = (wid // 2) * (2 * _TPW)
    off = (wid % 2) * _TPW

    pltpu.sync_copy(pos_hbm.at[pl.ds(t0, _TPW)], pos_v)
    pltpu.sync_copy(x_hbm.at[:, pl.ds(c0, 2 * _TPW)], idx_v)

    for s in range(_NBUF):
        pltpu.async_copy(tok_hbm.at[idx_v.at[s, pl.ds(off, _TPW)]],
                         gbuf.at[s], gsems[s])

    def group(i, carry):
        for s in range(_NBUF):
            b = i * _NBUF + s
            pltpu.make_async_copy(tok_hbm.at[idx_v.at[s, pl.ds(off, _TPW)]],
                                  gbuf.at[s], gsems[s]).wait()
            @pl.when(i > 0)
            def _wait_store():
                pltpu.make_async_copy(obuf.at[s],
                                      out_hbm.at[b - _NBUF, pl.ds(t0, _TPW)],
                                      ssems[s]).wait()

            def fma_row(r, c):
                for j in range(_HID // _LANES):
                    sl = pl.ds(j * _LANES, _LANES)
                    obuf[s, r, sl] = gbuf[s, r, sl] * _SCALE + pos_v[r, sl]
                return c

            lax.fori_loop(0, _TPW, fma_row, 0)
            pltpu.async_copy(obuf.at[s], out_hbm.at[b, pl.ds(t0, _TPW)],
                             ssems[s])
            @pl.when(i < _BATCH // _NBUF - 1)
            def _next_gather():
                pltpu.async_copy(
                    tok_hbm.at[idx_v.at[b + _NBUF, pl.ds(off, _TPW)]],
                    gbuf.at[s], gsems[s])
        return carry

    lax.fori_loop(0, _BATCH // _NBUF, group, 0)

    for s in range(_NBUF):
        pltpu.make_async_copy(obuf.at[s],
                              out_hbm.at[_BATCH - _NBUF + s, pl.ds(t0, _TPW)],
                              ssems[s]).wait()


@jax.jit
def kernel(x, token_table, pos_table):
    x = x.astype(jnp.int32)
    mesh = plsc.VectorSubcoreMesh(core_axis_name="c", subcore_axis_name="s")
    f = functools.partial(
        pl.kernel,
        mesh=mesh,
        out_type=jax.ShapeDtypeStruct((_BATCH, _MAXLEN, _HID), jnp.float32),
        scratch_types=[
            pltpu.VMEM((_BATCH, 2 * _TPW), jnp.int32),
            pltpu.VMEM((_NBUF, _TPW, _HID), jnp.float32),
            pltpu.VMEM((_NBUF, _TPW, _HID), jnp.float32),
            pltpu.VMEM((_TPW, _HID), jnp.float32),
            [pltpu.SemaphoreType.DMA] * _NBUF,
            [pltpu.SemaphoreType.DMA] * _NBUF,
        ],
    )(_body)
    return f(x, token_table, pos_table)
